# baseline (speedup 1.0000x reference)
import jax
import jax.numpy as jnp
import numpy as np
from jax.experimental import pallas as pl

_NUM_BUCKETS = 512
_MAX_DIST = 3.0
_INV_RANGE = float(np.float32(1.0) / np.float32(2.0 * _MAX_DIST))


def _body(xy_ref, out_ref):
    seq = xy_ref.shape[-1]
    j = pl.program_id(1)
    half = j // 2
    part = j % 2
    v = jnp.where(half == 0, xy_ref[0, 0:1, :], xy_ref[0, 1:2, :])
    lbl = jnp.clip(
        ((v * _INV_RANGE + 0.5) * _NUM_BUCKETS).astype(jnp.int32),
        0, _NUM_BUCKETS - 1)
    rows = jax.lax.broadcasted_iota(jnp.int32, (_NUM_BUCKETS // 2, seq), 0)
    out_ref[0] = (rows == (lbl - part * (_NUM_BUCKETS // 2))).astype(jnp.float32)


def kernel(xy):
    bs, _, seq = xy.shape
    return pl.pallas_call(
        _body,
        grid=(bs, 4),
        in_specs=[pl.BlockSpec((1, 2, seq), lambda b, j: (b, 0, 0))],
        out_specs=pl.BlockSpec((1, _NUM_BUCKETS // 2, seq), lambda b, j: (b, j, 0)),
        out_shape=jax.ShapeDtypeStruct((bs, 2 * _NUM_BUCKETS, seq), jnp.float32),
    )(xy)


# speedup vs baseline: 1.3086x; 1.3086x over previous
import jax
import jax.numpy as jnp
import numpy as np
from jax.experimental import pallas as pl

_NUM_BUCKETS = 512
_MAX_DIST = 3.0
_INV_RANGE = float(np.float32(1.0) / np.float32(2.0 * _MAX_DIST))


def _body(xy_ref, out_ref):
    seq = xy_ref.shape[-1]
    j = pl.program_id(1)
    v = jnp.where(j == 0, xy_ref[0, 0:1, :], xy_ref[0, 1:2, :])
    lbl = jnp.clip(
        ((v * _INV_RANGE + 0.5) * _NUM_BUCKETS).astype(jnp.int32),
        0, _NUM_BUCKETS - 1)
    rows = jax.lax.broadcasted_iota(jnp.int32, (_NUM_BUCKETS, seq), 0)
    out_ref[0] = (rows == lbl).astype(jnp.float32)


def kernel(xy):
    bs, _, seq = xy.shape
    return pl.pallas_call(
        _body,
        grid=(bs, 2),
        in_specs=[pl.BlockSpec((1, 2, seq), lambda b, j: (b, 0, 0))],
        out_specs=pl.BlockSpec((1, _NUM_BUCKETS, seq), lambda b, j: (b, j, 0)),
        out_shape=jax.ShapeDtypeStruct((bs, 2 * _NUM_BUCKETS, seq), jnp.float32),
    )(xy)
